# baseline (speedup 1.0000x reference)
import functools

import jax
import jax.numpy as jnp
from jax import lax
from jax.experimental import pallas as pl
from jax.experimental.pallas import tpu as pltpu
from jax.experimental.pallas import tpu_sc as plsc

N_PLAYERS = 1000000
N_COURSES = 100000
EMBED_DIM = 16
BATCH = 16384

NC = 2
NS = 16
NW = NC * NS
BPW = BATCH // NW
CHUNK = 128
NCHUNK = BPW // CHUNK
GROUPS = BPW // 16

_mesh = plsc.VectorSubcoreMesh(core_axis_name="c", subcore_axis_name="s")


@functools.partial(
    pl.kernel,
    out_type=jax.ShapeDtypeStruct((BATCH,), jnp.float32),
    mesh=_mesh,
    compiler_params=pltpu.CompilerParams(
        needs_layout_passes=False, use_tc_tiling_on_sc=False),
    scratch_types=[
        pltpu.VMEM((NCHUNK, CHUNK), jnp.int32),
        pltpu.VMEM((NCHUNK, CHUNK), jnp.int32),
        pltpu.VMEM((EMBED_DIM * BPW,), jnp.float32),
        pltpu.VMEM((EMBED_DIM * BPW,), jnp.float32),
        pltpu.VMEM((BPW,), jnp.float32),
        pltpu.VMEM((BPW,), jnp.float32),
        pltpu.VMEM((16,), jnp.float32),
        pltpu.VMEM((BPW,), jnp.float32),
        pltpu.SemaphoreType.DMA,
    ],
)
def _sc_kernel(pid_hbm, cid_hbm, pemb_hbm, cemb_hbm, pbias_hbm, cbias_hbm,
               gbias_hbm, out_hbm,
               idx_p, idx_c, prow, crow, pbv, cbv, gbv, outv, sem):
    wid = lax.axis_index("s") * NC + lax.axis_index("c")

    pltpu.sync_copy(pid_hbm.at[wid], idx_p)
    pltpu.sync_copy(cid_hbm.at[wid], idx_c)
    pltpu.sync_copy(gbias_hbm, gbv)

    copies = []
    for j in range(NCHUNK):
        sl = pl.ds(j * CHUNK, CHUNK)
        copies.append(pltpu.async_copy(pbias_hbm.at[idx_p.at[j]], pbv.at[sl], sem))
        copies.append(pltpu.async_copy(cbias_hbm.at[idx_c.at[j]], cbv.at[sl], sem))

    for d in range(EMBED_DIM):
        p_slab = pemb_hbm.at[pl.ds(d * N_PLAYERS, N_PLAYERS)]
        c_slab = cemb_hbm.at[pl.ds(d * N_COURSES, N_COURSES)]
        for j in range(NCHUNK):
            dsl = pl.ds(d * BPW + j * CHUNK, CHUNK)
            copies.append(pltpu.async_copy(p_slab.at[idx_p.at[j]], prow.at[dsl], sem))
            copies.append(pltpu.async_copy(c_slab.at[idx_c.at[j]], crow.at[dsl], sem))
    for c in copies:
        c.wait()

    gb = gbv[...]

    def group_body(g, _):
        base = pl.multiple_of(g * 16, 16)
        acc = pbv[pl.ds(base, 16)] + cbv[pl.ds(base, 16)] + gb
        for d in range(EMBED_DIM):
            pv = prow[pl.ds(pl.multiple_of(d * BPW + g * 16, 16), 16)]
            cv = crow[pl.ds(pl.multiple_of(d * BPW + g * 16, 16), 16)]
            acc = acc + pv * cv
        outv[pl.ds(base, 16)] = 1.0 / (1.0 + jnp.exp(-acc))
        return 0

    lax.fori_loop(0, GROUPS, group_body, 0)

    base = pl.multiple_of(wid * BPW, BPW)
    pltpu.sync_copy(outv, out_hbm.at[pl.ds(base, BPW)])


def kernel(player_ids, course_ids, player_embed, course_embed, player_bias,
           course_bias, global_bias):
    pid = player_ids.astype(jnp.int32).reshape(NW, NCHUNK, CHUNK)
    cid = course_ids.astype(jnp.int32).reshape(NW, NCHUNK, CHUNK)
    pemb = player_embed.T.reshape(EMBED_DIM * N_PLAYERS)
    cemb = course_embed.T.reshape(EMBED_DIM * N_COURSES)
    pb = player_bias.reshape(N_PLAYERS)
    cb = course_bias.reshape(N_COURSES)
    gb = jnp.broadcast_to(global_bias.astype(jnp.float32), (16,))
    return _sc_kernel(pid, cid, pemb, cemb, pb, cb, gb)


# speedup vs baseline: 10.1646x; 10.1646x over previous
import functools

import jax
import jax.numpy as jnp
from jax import lax
from jax.experimental import pallas as pl
from jax.experimental.pallas import tpu as pltpu
from jax.experimental.pallas import tpu_sc as plsc

N_PLAYERS = 1000000
N_COURSES = 100000
EMBED_DIM = 16
BATCH = 16384

NC = 2
NS = 16
NW = NC * NS
BPW = BATCH // NW
CHUNK = 128
NCHUNK = BPW // CHUNK
GROUPS = BPW // 16

P_TILES = (N_PLAYERS + 127) // 128
C_TILES = (N_COURSES + 127) // 128
P_PAD = P_TILES * 128 - N_PLAYERS
C_PAD = C_TILES * 128 - N_COURSES
P_SLAB = P_TILES * 1024
C_SLAB = C_TILES * 1024
P_LEN = (P_TILES - 1) * 1024 + 128
C_LEN = (C_TILES - 1) * 1024 + 128

_mesh = plsc.VectorSubcoreMesh(core_axis_name="c", subcore_axis_name="s")


@functools.partial(
    pl.kernel,
    out_type=jax.ShapeDtypeStruct((BATCH,), jnp.float32),
    mesh=_mesh,
    compiler_params=pltpu.CompilerParams(
        needs_layout_passes=False, use_tc_tiling_on_sc=False),
    scratch_types=[
        pltpu.VMEM((BPW,), jnp.int32),
        pltpu.VMEM((BPW,), jnp.int32),
        pltpu.VMEM((BPW,), jnp.int32),
        pltpu.VMEM((BPW,), jnp.int32),
        pltpu.VMEM((EMBED_DIM * BPW,), jnp.float32),
        pltpu.VMEM((EMBED_DIM * BPW,), jnp.float32),
        pltpu.VMEM((BPW,), jnp.float32),
        pltpu.VMEM((BPW,), jnp.float32),
        pltpu.VMEM((16,), jnp.float32),
        pltpu.VMEM((BPW,), jnp.float32),
        pltpu.SemaphoreType.DMA,
    ],
)
def _sc_kernel(pid_hbm, cid_hbm, pemb_hbm, cemb_hbm, pbias_hbm, cbias_hbm,
               gbias_hbm, out_hbm,
               idx_p, idx_c, up, uc, prow, crow, pbv, cbv, gbv, outv, sem):
    wid = lax.axis_index("s") * NC + lax.axis_index("c")

    pltpu.sync_copy(pid_hbm.at[wid], idx_p)
    pltpu.sync_copy(cid_hbm.at[wid], idx_c)
    pltpu.sync_copy(gbias_hbm, gbv)

    def off_body(m, _):
        sl = pl.ds(pl.multiple_of(m * 16, 16), 16)
        pi = idx_p[sl]
        ci = idx_c[sl]
        up[sl] = ((pi >> 7) << 10) + (pi & 127)
        uc[sl] = ((ci >> 7) << 10) + (ci & 127)
        return 0

    lax.fori_loop(0, GROUPS, off_body, 0)

    copies = []
    for j in range(NCHUNK):
        sl = pl.ds(j * CHUNK, CHUNK)
        copies.append(pltpu.async_copy(pbias_hbm.at[idx_p.at[sl]], pbv.at[sl], sem))
        copies.append(pltpu.async_copy(cbias_hbm.at[idx_c.at[sl]], cbv.at[sl], sem))

    for d in range(EMBED_DIM):
        p_slab = pemb_hbm.at[pl.ds((d // 8) * P_SLAB + (d % 8) * 128, P_LEN)]
        c_slab = cemb_hbm.at[pl.ds((d // 8) * C_SLAB + (d % 8) * 128, C_LEN)]
        for j in range(NCHUNK):
            sl = pl.ds(j * CHUNK, CHUNK)
            dsl = pl.ds(d * BPW + j * CHUNK, CHUNK)
            copies.append(pltpu.async_copy(p_slab.at[up.at[sl]], prow.at[dsl], sem))
            copies.append(pltpu.async_copy(c_slab.at[uc.at[sl]], crow.at[dsl], sem))
    for c in copies:
        c.wait()

    gb = gbv[...]

    def group_body(g, _):
        base = pl.multiple_of(g * 16, 16)
        acc = pbv[pl.ds(base, 16)] + cbv[pl.ds(base, 16)] + gb
        for d in range(EMBED_DIM):
            pv = prow[pl.ds(pl.multiple_of(d * BPW + g * 16, 16), 16)]
            cv = crow[pl.ds(pl.multiple_of(d * BPW + g * 16, 16), 16)]
            acc = acc + pv * cv
        outv[pl.ds(base, 16)] = 1.0 / (1.0 + jnp.exp(-acc))
        return 0

    lax.fori_loop(0, GROUPS, group_body, 0)

    base = pl.multiple_of(wid * BPW, BPW)
    pltpu.sync_copy(outv, out_hbm.at[pl.ds(base, BPW)])


def _physical_flat(table, n_rows, n_tiles, pad):
    t = jnp.pad(table.T, ((0, 0), (0, pad)))
    return t.reshape(2, 8, n_tiles, 128).transpose(0, 2, 1, 3).reshape(-1)


def kernel(player_ids, course_ids, player_embed, course_embed, player_bias,
           course_bias, global_bias):
    pid = player_ids.astype(jnp.int32).reshape(NW, BPW)
    cid = course_ids.astype(jnp.int32).reshape(NW, BPW)
    pemb = _physical_flat(player_embed, N_PLAYERS, P_TILES, P_PAD)
    cemb = _physical_flat(course_embed, N_COURSES, C_TILES, C_PAD)
    pb = player_bias.reshape(N_PLAYERS)
    cb = course_bias.reshape(N_COURSES)
    gb = jnp.broadcast_to(global_bias.astype(jnp.float32), (16,))
    return _sc_kernel(pid, cid, pemb, cemb, pb, cb, gb)


# speedup vs baseline: 14.1853x; 1.3956x over previous
import functools

import jax
import jax.numpy as jnp
from jax import lax
from jax.experimental import pallas as pl
from jax.experimental.pallas import tpu as pltpu
from jax.experimental.pallas import tpu_sc as plsc

N_PLAYERS = 1000000
N_COURSES = 100000
EMBED_DIM = 16
BATCH = 16384

NC = 2
NS = 16
NW = NC * NS
BPW = BATCH // NW
CHUNK = 128
NCHUNK = BPW // CHUNK
GROUPS = BPW // 16

P_TILES = (N_PLAYERS + 127) // 128
C_TILES = (N_COURSES + 127) // 128
P_PAD = P_TILES * 128 - N_PLAYERS
C_PAD = C_TILES * 128 - N_COURSES
P_SLAB = P_TILES * 1024
C_SLAB = C_TILES * 1024
P_LEN = (P_TILES - 1) * 1024 + 128
C_LEN = (C_TILES - 1) * 1024 + 128

_mesh = plsc.VectorSubcoreMesh(core_axis_name="c", subcore_axis_name="s")


@functools.partial(
    pl.kernel,
    out_type=jax.ShapeDtypeStruct((BATCH,), jnp.float32),
    mesh=_mesh,
    compiler_params=pltpu.CompilerParams(
        needs_layout_passes=False, use_tc_tiling_on_sc=False),
    scratch_types=[
        pltpu.VMEM((BPW,), jnp.int32),
        pltpu.VMEM((BPW,), jnp.int32),
        pltpu.VMEM((BPW,), jnp.int32),
        pltpu.VMEM((BPW,), jnp.int32),
        pltpu.VMEM((EMBED_DIM * BPW,), jnp.float32),
        pltpu.VMEM((EMBED_DIM * BPW,), jnp.float32),
        pltpu.VMEM((BPW,), jnp.float32),
        pltpu.VMEM((BPW,), jnp.float32),
        pltpu.VMEM((16,), jnp.float32),
        pltpu.VMEM((BPW,), jnp.float32),
        pltpu.SemaphoreType.DMA,
    ],
)
def _sc_kernel(pid_hbm, cid_hbm, pemb_hbm, cemb_hbm, pbias_hbm, cbias_hbm,
               gbias_hbm, out_hbm,
               idx_p, idx_c, up, uc, prow, crow, pbv, cbv, gbv, outv, sem):
    wid = lax.axis_index("s") * NC + lax.axis_index("c")

    pltpu.sync_copy(pid_hbm.at[wid], idx_p)
    pltpu.sync_copy(cid_hbm.at[wid], idx_c)
    pltpu.sync_copy(gbias_hbm, gbv)

    def off_body(m, _):
        sl = pl.ds(pl.multiple_of(m * 16, 16), 16)
        pi = idx_p[sl]
        ci = idx_c[sl]
        up[sl] = ((pi >> 7) << 10) + (pi & 127)
        uc[sl] = ((ci >> 7) << 10) + (ci & 127)
        return 0

    lax.fori_loop(0, GROUPS, off_body, 0)

    copies = []
    for j in range(NCHUNK):
        sl = pl.ds(j * CHUNK, CHUNK)
        copies.append(pltpu.async_copy(pbias_hbm.at[idx_p.at[sl]], pbv.at[sl], sem))
        copies.append(pltpu.async_copy(cbias_hbm.at[idx_c.at[sl]], cbv.at[sl], sem))

    for d in range(EMBED_DIM):
        p_slab = pemb_hbm.at[pl.ds((d // 8) * P_SLAB + (d % 8) * 128, P_LEN)]
        c_slab = cemb_hbm.at[pl.ds((d // 8) * C_SLAB + (d % 8) * 128, C_LEN)]
        for j in range(NCHUNK):
            sl = pl.ds(j * CHUNK, CHUNK)
            dsl = pl.ds(d * BPW + j * CHUNK, CHUNK)
            copies.append(pltpu.async_copy(p_slab.at[up.at[sl]], prow.at[dsl], sem))
            copies.append(pltpu.async_copy(c_slab.at[uc.at[sl]], crow.at[dsl], sem))
    for c in copies:
        c.wait()

    gb = gbv[...]

    def group_body(g, _):
        base = pl.multiple_of(g * 16, 16)
        acc = pbv[pl.ds(base, 16)] + cbv[pl.ds(base, 16)] + gb
        for d in range(EMBED_DIM):
            pv = prow[pl.ds(pl.multiple_of(d * BPW + g * 16, 16), 16)]
            cv = crow[pl.ds(pl.multiple_of(d * BPW + g * 16, 16), 16)]
            acc = acc + pv * cv
        outv[pl.ds(base, 16)] = 1.0 / (1.0 + jnp.exp(-acc))
        return 0

    lax.fori_loop(0, GROUPS, group_body, 0)

    base = pl.multiple_of(wid * BPW, BPW)
    pltpu.sync_copy(outv, out_hbm.at[pl.ds(base, BPW)])


def _bias_flat(bias, granule=1024):
    n = bias.shape[0]
    padded = ((n + granule - 1) // granule) * granule
    return jnp.pad(bias, ((0, padded - n), (0, 0))).T.reshape(-1)


def _physical_flat(table, n_rows, n_tiles, pad):
    t = jnp.pad(table.T, ((0, 0), (0, pad)))
    return t.reshape(2, 8, n_tiles, 128).transpose(0, 2, 1, 3).reshape(-1)


def kernel(player_ids, course_ids, player_embed, course_embed, player_bias,
           course_bias, global_bias):
    pid = player_ids.astype(jnp.int32).reshape(NW, BPW)
    cid = course_ids.astype(jnp.int32).reshape(NW, BPW)
    pemb = _physical_flat(player_embed, N_PLAYERS, P_TILES, P_PAD)
    cemb = _physical_flat(course_embed, N_COURSES, C_TILES, C_PAD)
    pb = _bias_flat(player_bias)
    cb = _bias_flat(course_bias)
    gb = jnp.broadcast_to(global_bias.astype(jnp.float32), (16,))
    return _sc_kernel(pid, cid, pemb, cemb, pb, cb, gb)
